# baseline (speedup 1.0000x reference)
import functools

import jax
import jax.numpy as jnp
from jax import lax
from jax.experimental import pallas as pl
from jax.experimental.pallas import tpu as pltpu
from jax.experimental.pallas import tpu_sc as plsc

N = 20000
MAX_OUT = 100
IOU_THRESH = 0.7

L = 16
NSUB = 16
NPAD = 20480
SHARD = NPAD // NSUB
NSL = SHARD // L
OUTPAD = 512
ROW = 16


def _nms_sc_body(x1h, y1h, x2h, y2h, sch, outh,
                 x1v, y1v, x2v, y2v, scv, arv, wkv, ixv,
                 tbl_sh, tblv, stage, sci, outv):
    sid = lax.axis_index("s")
    cid = lax.axis_index("c")
    base = sid * SHARD

    pltpu.sync_copy(x1h.at[pl.ds(base, SHARD)], x1v)
    pltpu.sync_copy(y1h.at[pl.ds(base, SHARD)], y1v)
    pltpu.sync_copy(x2h.at[pl.ds(base, SHARD)], x2v)
    pltpu.sync_copy(y2h.at[pl.ds(base, SHARD)], y2v)
    pltpu.sync_copy(sch.at[pl.ds(base, SHARD)], scv)

    ninf = jnp.float32(-jnp.inf)
    bigf = jnp.float32(3.0e38)
    ii = lax.iota(jnp.int32, L)
    iif = ii.astype(jnp.float32)
    base_f = jnp.broadcast_to(base, (L,)).astype(jnp.float32)
    bv0 = jnp.broadcast_to(ninf, (L,))
    bi0 = base_f + iif

    def upd2(w, ixs, st):
        bv1, bi1, bv2, bi2 = st
        m1 = w > bv1
        dv = jnp.where(m1, bv1, w)
        di = jnp.where(m1, bi1, ixs)
        nbv1 = jnp.where(m1, w, bv1)
        nbi1 = jnp.where(m1, ixs, bi1)
        m2 = (dv > bv2) | ((dv == bv2) & (di < bi2))
        return nbv1, nbi1, jnp.where(m2, dv, bv2), jnp.where(m2, di, bi2)

    def xlane(v, ix):
        m = jnp.max(v, axis=0)
        i = jnp.min(jnp.where(v == m, ix, bigf), axis=0)
        return jnp.broadcast_to(m, (L,)), jnp.broadcast_to(i, (L,))

    def init_j(j, st):
        s = pl.ds(j * L, L)
        a, b, c, d, sc = x1v[s], y1v[s], x2v[s], y2v[s], scv[s]
        arv[s] = jnp.maximum(c - a, 0.0) * jnp.maximum(d - b, 0.0)
        valid = (c > a + 1.0) & (d > b + 1.0)
        w = jnp.where(valid, sc, ninf)
        wkv[s] = w
        ix = base_f + jnp.broadcast_to(j * L, (L,)).astype(jnp.float32) + iif
        ixv[s] = ix
        return upd2(w, ix, st)

    st0 = (bv0, bi0, bv0, bi0)
    top0 = lax.fori_loop(0, NSL, init_j, st0)

    def cond_fn(c):
        return c[0] < MAX_OUT

    def round_body(c):
        oi, r, bv1, bi1, bv2, bi2 = c
        l1v, l1i = xlane(bv1, bi1)
        repl = bi1 == l1i
        l2v, l2i = xlane(jnp.where(repl, bv2, bv1), jnp.where(repl, bi2, bi1))
        loc1 = (l1i - base_f).astype(jnp.int32)
        loc2 = (l2i - base_f).astype(jnp.int32)
        pub = jnp.where(ii == 0, l1v, jnp.float32(0.0))
        pub = jnp.where(ii == 1, l1i, pub)
        pub = jnp.where(ii == 2, plsc.load_gather(x1v, [loc1]), pub)
        pub = jnp.where(ii == 3, plsc.load_gather(y1v, [loc1]), pub)
        pub = jnp.where(ii == 4, plsc.load_gather(x2v, [loc1]), pub)
        pub = jnp.where(ii == 5, plsc.load_gather(y2v, [loc1]), pub)
        pub = jnp.where(ii == 6, plsc.load_gather(scv, [loc1]), pub)
        pub = jnp.where(ii == 7, l2v, pub)
        pub = jnp.where(ii == 8, l2i, pub)
        pub = jnp.where(ii == 9, plsc.load_gather(x1v, [loc2]), pub)
        pub = jnp.where(ii == 10, plsc.load_gather(y1v, [loc2]), pub)
        pub = jnp.where(ii == 11, plsc.load_gather(x2v, [loc2]), pub)
        pub = jnp.where(ii == 12, plsc.load_gather(y2v, [loc2]), pub)
        pub = jnp.where(ii == 13, plsc.load_gather(scv, [loc2]), pub)
        stage[...] = pub
        off = (r & 1) * (NSUB * ROW)
        pltpu.sync_copy(stage, tbl_sh.at[pl.ds(off + sid * ROW, ROW)])
        plsc.subcore_barrier()
        pltpu.sync_copy(tbl_sh.at[pl.ds(off, NSUB * ROW)], tblv)

        rows = ii * ROW
        v1 = plsc.load_gather(tblv, [rows])
        i1 = plsc.load_gather(tblv, [rows + 1])
        g1v, g1i = xlane(v1, i1)
        w1 = (g1i.astype(jnp.int32) // SHARD) * ROW
        bx1 = plsc.load_gather(tblv, [w1 + 2])
        by1 = plsc.load_gather(tblv, [w1 + 3])
        bx2 = plsc.load_gather(tblv, [w1 + 4])
        by2 = plsc.load_gather(tblv, [w1 + 5])
        bsc = plsc.load_gather(tblv, [w1 + 6])
        ar1 = jnp.maximum(bx2 - bx1, 0.0) * jnp.maximum(by2 - by1, 0.0)

        repl2 = i1 == g1i
        v2c = jnp.where(repl2, plsc.load_gather(tblv, [rows + 7]), v1)
        i2c = jnp.where(repl2, plsc.load_gather(tblv, [rows + 8]), i1)
        g2v, g2i = xlane(v2c, i2c)
        w2 = (g2i.astype(jnp.int32) // SHARD) * ROW
        own1i = plsc.load_gather(tblv, [w2 + 1])
        cofs = jnp.where(own1i == g2i, 2, 9)
        cx1 = plsc.load_gather(tblv, [w2 + cofs])
        cy1 = plsc.load_gather(tblv, [w2 + cofs + 1])
        cx2 = plsc.load_gather(tblv, [w2 + cofs + 2])
        cy2 = plsc.load_gather(tblv, [w2 + cofs + 3])
        csc = plsc.load_gather(tblv, [w2 + cofs + 4])
        ar2 = jnp.maximum(cx2 - cx1, 0.0) * jnp.maximum(cy2 - cy1, 0.0)

        px1 = jnp.maximum(cx1, bx1)
        py1 = jnp.maximum(cy1, by1)
        px2 = jnp.minimum(cx2, bx2)
        py2 = jnp.minimum(cy2, by2)
        pint = jnp.maximum(px2 - px1, 0.0) * jnp.maximum(py2 - py1, 0.0)
        piou = pint / (ar2 + ar1 - pint + 1e-8)
        acc = jnp.logical_not(piou > IOU_THRESH) & (g2v != ninf)

        def sup_one(s, st2):
            xa, ya, xb, yb = x1v[s], y1v[s], x2v[s], y2v[s]
            ar, ixs = arv[s], ixv[s]
            qx1 = jnp.maximum(xa, bx1)
            qy1 = jnp.maximum(ya, by1)
            qx2 = jnp.minimum(xb, bx2)
            qy2 = jnp.minimum(yb, by2)
            qin = jnp.maximum(qx2 - qx1, 0.0) * jnp.maximum(qy2 - qy1, 0.0)
            sup1 = qin / (ar + ar1 - qin + 1e-8) > IOU_THRESH
            ux1 = jnp.maximum(xa, cx1)
            uy1 = jnp.maximum(ya, cy1)
            ux2 = jnp.minimum(xb, cx2)
            uy2 = jnp.minimum(yb, cy2)
            uin = jnp.maximum(ux2 - ux1, 0.0) * jnp.maximum(uy2 - uy1, 0.0)
            sup2 = acc & (uin / (ar + ar2 - uin + 1e-8) > IOU_THRESH)
            w = jnp.where(sup1 | sup2, ninf, wkv[s])
            wkv[s] = w
            return upd2(w, ixs, st2)

        def sup_j(j, st2):
            sta, stb, stc, std = st2
            sta = sup_one(pl.ds(j * 4 * L, L), sta)
            stb = sup_one(pl.ds(j * 4 * L + L, L), stb)
            stc = sup_one(pl.ds(j * 4 * L + 2 * L, L), stc)
            std = sup_one(pl.ds(j * 4 * L + 3 * L, L), std)
            return sta, stb, stc, std

        sta, stb, stc, std = lax.fori_loop(
            0, NSL // 4, sup_j, (st0, st0, st0, st0))
        nst = upd2(stb[0], stb[1], sta)
        nst = upd2(stb[2], stb[3], nst)
        for other in (stc, std):
            nst = upd2(other[0], other[1], nst)
            nst = upd2(other[2], other[3], nst)

        z = jnp.float32(0.0)
        row1 = jnp.where(ii == 0, bx1, z)
        row1 = jnp.where(ii == 1, by1, row1)
        row1 = jnp.where(ii == 2, bx2, row1)
        row1 = jnp.where(ii == 3, by2, row1)
        row1 = jnp.where(ii == 4, bsc, row1)
        oiv = jnp.broadcast_to(oi, (L,))
        plsc.store_scatter(outv, [oiv * 5 + ii], row1, mask=ii < 5)
        row2 = jnp.where(ii == 0, cx1, z)
        row2 = jnp.where(ii == 1, cy1, row2)
        row2 = jnp.where(ii == 2, cx2, row2)
        row2 = jnp.where(ii == 3, cy2, row2)
        row2 = jnp.where(ii == 4, csc, row2)
        plsc.store_scatter(outv, [(oiv + 1) * 5 + ii], row2, mask=(ii < 5) & acc)

        sci[...] = jnp.where(acc, 1, 0)
        noi = oi + 1 + sci[...][0]
        return (noi, r + 1) + nst

    lax.while_loop(cond_fn, round_body,
                   (jnp.int32(0), jnp.int32(0)) + top0)

    @pl.when((sid == 0) & (cid == 0))
    def _():
        pltpu.sync_copy(outv, outh)


def _make_nms_sc():
    mesh = plsc.VectorSubcoreMesh(
        core_axis_name="c", subcore_axis_name="s", num_cores=2, num_subcores=NSUB
    )
    return pl.kernel(
        _nms_sc_body,
        out_type=jax.ShapeDtypeStruct((OUTPAD,), jnp.float32),
        mesh=mesh,
        compiler_params=pltpu.CompilerParams(needs_layout_passes=False),
        scratch_types=[
            pltpu.VMEM((SHARD,), jnp.float32),
            pltpu.VMEM((SHARD,), jnp.float32),
            pltpu.VMEM((SHARD,), jnp.float32),
            pltpu.VMEM((SHARD,), jnp.float32),
            pltpu.VMEM((SHARD,), jnp.float32),
            pltpu.VMEM((SHARD,), jnp.float32),
            pltpu.VMEM((SHARD,), jnp.float32),
            pltpu.VMEM((SHARD,), jnp.float32),
            pltpu.VMEM_SHARED((2 * NSUB * ROW,), jnp.float32),
            pltpu.VMEM((NSUB * ROW,), jnp.float32),
            pltpu.VMEM((ROW,), jnp.float32),
            pltpu.VMEM((L,), jnp.int32),
            pltpu.VMEM((OUTPAD,), jnp.float32),
        ],
    )


def kernel(boxes, scores):
    pad = NPAD - N
    x1 = jnp.pad(boxes[:, 0], (0, pad))
    y1 = jnp.pad(boxes[:, 1], (0, pad))
    x2 = jnp.pad(boxes[:, 2], (0, pad))
    y2 = jnp.pad(boxes[:, 3], (0, pad))
    sc = jnp.pad(scores, (0, pad))
    out = _make_nms_sc()(x1, y1, x2, y2, sc)
    return out[: MAX_OUT * 5].reshape(MAX_OUT, 5)


# speedup vs baseline: 1.0572x; 1.0572x over previous
import functools

import jax
import jax.numpy as jnp
from jax import lax
from jax.experimental import pallas as pl
from jax.experimental.pallas import tpu as pltpu
from jax.experimental.pallas import tpu_sc as plsc

N = 20000
MAX_OUT = 100
IOU_THRESH = 0.7

L = 16
NSUB = 16
NPAD = 20480
SHARD = NPAD // NSUB
NSL = SHARD // L
OUTPAD = 512
ROW = 16


def _nms_sc_body(x1h, y1h, x2h, y2h, sch, outh,
                 x1v, y1v, x2v, y2v, scv, arv, wkv, ixv,
                 tbl_sh, tblv, stage, sci, outv):
    sid = lax.axis_index("s")
    cid = lax.axis_index("c")
    base = sid * SHARD

    pltpu.sync_copy(x1h.at[pl.ds(base, SHARD)], x1v)
    pltpu.sync_copy(y1h.at[pl.ds(base, SHARD)], y1v)
    pltpu.sync_copy(x2h.at[pl.ds(base, SHARD)], x2v)
    pltpu.sync_copy(y2h.at[pl.ds(base, SHARD)], y2v)
    pltpu.sync_copy(sch.at[pl.ds(base, SHARD)], scv)

    ninf = jnp.float32(-jnp.inf)
    bigf = jnp.float32(3.0e38)
    ii = lax.iota(jnp.int32, L)
    iif = ii.astype(jnp.float32)
    base_f = jnp.broadcast_to(base, (L,)).astype(jnp.float32)
    bv0 = jnp.broadcast_to(ninf, (L,))
    bi0 = base_f + iif

    def upd2(w, ixs, st):
        bv1, bi1, bv2, bi2 = st
        m1 = w > bv1
        dv = jnp.where(m1, bv1, w)
        di = jnp.where(m1, bi1, ixs)
        nbv1 = jnp.where(m1, w, bv1)
        nbi1 = jnp.where(m1, ixs, bi1)
        m2 = (dv > bv2) | ((dv == bv2) & (di < bi2))
        return nbv1, nbi1, jnp.where(m2, dv, bv2), jnp.where(m2, di, bi2)

    def xlane(v, ix):
        m = jnp.max(v, axis=0)
        i = jnp.min(jnp.where(v == m, ix, bigf), axis=0)
        return jnp.broadcast_to(m, (L,)), jnp.broadcast_to(i, (L,))

    def init_j(j, st):
        s = pl.ds(j * L, L)
        a, b, c, d, sc = x1v[s], y1v[s], x2v[s], y2v[s], scv[s]
        arv[s] = jnp.maximum(c - a, 0.0) * jnp.maximum(d - b, 0.0)
        valid = (c > a + 1.0) & (d > b + 1.0)
        w = jnp.where(valid, sc, ninf)
        wkv[s] = w
        ix = base_f + jnp.broadcast_to(j * L, (L,)).astype(jnp.float32) + iif
        ixv[s] = ix
        return upd2(w, ix, st)

    st0 = (bv0, bi0, bv0, bi0)
    top0 = lax.fori_loop(0, NSL, init_j, st0)

    def cond_fn(c):
        return c[0] < MAX_OUT

    def round_body(c):
        oi, r, bv1, bi1, bv2, bi2 = c
        l1v, l1i = xlane(bv1, bi1)
        repl = bi1 == l1i
        l2v, l2i = xlane(jnp.where(repl, bv2, bv1), jnp.where(repl, bi2, bi1))
        loc1 = (l1i - base_f).astype(jnp.int32)
        loc2 = (l2i - base_f).astype(jnp.int32)
        pub = jnp.where(ii == 0, l1v, jnp.float32(0.0))
        pub = jnp.where(ii == 1, l1i, pub)
        pub = jnp.where(ii == 2, plsc.load_gather(x1v, [loc1]), pub)
        pub = jnp.where(ii == 3, plsc.load_gather(y1v, [loc1]), pub)
        pub = jnp.where(ii == 4, plsc.load_gather(x2v, [loc1]), pub)
        pub = jnp.where(ii == 5, plsc.load_gather(y2v, [loc1]), pub)
        pub = jnp.where(ii == 6, plsc.load_gather(scv, [loc1]), pub)
        pub = jnp.where(ii == 7, l2v, pub)
        pub = jnp.where(ii == 8, l2i, pub)
        pub = jnp.where(ii == 9, plsc.load_gather(x1v, [loc2]), pub)
        pub = jnp.where(ii == 10, plsc.load_gather(y1v, [loc2]), pub)
        pub = jnp.where(ii == 11, plsc.load_gather(x2v, [loc2]), pub)
        pub = jnp.where(ii == 12, plsc.load_gather(y2v, [loc2]), pub)
        pub = jnp.where(ii == 13, plsc.load_gather(scv, [loc2]), pub)
        stage[...] = pub
        off = (r & 1) * (NSUB * ROW)
        pltpu.sync_copy(stage, tbl_sh.at[pl.ds(off + sid * ROW, ROW)])
        plsc.subcore_barrier()
        pltpu.sync_copy(tbl_sh.at[pl.ds(off, NSUB * ROW)], tblv)

        rows = ii * ROW
        v1 = plsc.load_gather(tblv, [rows])
        i1 = plsc.load_gather(tblv, [rows + 1])
        g1v, g1i = xlane(v1, i1)
        w1 = (g1i.astype(jnp.int32) // SHARD) * ROW
        bx1 = plsc.load_gather(tblv, [w1 + 2])
        by1 = plsc.load_gather(tblv, [w1 + 3])
        bx2 = plsc.load_gather(tblv, [w1 + 4])
        by2 = plsc.load_gather(tblv, [w1 + 5])
        bsc = plsc.load_gather(tblv, [w1 + 6])
        ar1 = jnp.maximum(bx2 - bx1, 0.0) * jnp.maximum(by2 - by1, 0.0)

        repl2 = i1 == g1i
        v2c = jnp.where(repl2, plsc.load_gather(tblv, [rows + 7]), v1)
        i2c = jnp.where(repl2, plsc.load_gather(tblv, [rows + 8]), i1)
        g2v, g2i = xlane(v2c, i2c)
        w2 = (g2i.astype(jnp.int32) // SHARD) * ROW
        own1i = plsc.load_gather(tblv, [w2 + 1])
        cofs = jnp.where(own1i == g2i, 2, 9)
        cx1 = plsc.load_gather(tblv, [w2 + cofs])
        cy1 = plsc.load_gather(tblv, [w2 + cofs + 1])
        cx2 = plsc.load_gather(tblv, [w2 + cofs + 2])
        cy2 = plsc.load_gather(tblv, [w2 + cofs + 3])
        csc = plsc.load_gather(tblv, [w2 + cofs + 4])
        ar2 = jnp.maximum(cx2 - cx1, 0.0) * jnp.maximum(cy2 - cy1, 0.0)

        px1 = jnp.maximum(cx1, bx1)
        py1 = jnp.maximum(cy1, by1)
        px2 = jnp.minimum(cx2, bx2)
        py2 = jnp.minimum(cy2, by2)
        pint = jnp.maximum(px2 - px1, 0.0) * jnp.maximum(py2 - py1, 0.0)
        piou = pint / (ar2 + ar1 - pint + 1e-8)
        acc = jnp.logical_not(piou > IOU_THRESH) & (g2v != ninf)

        def sup_one(s, st2):
            xa, ya, xb, yb = x1v[s], y1v[s], x2v[s], y2v[s]
            ar, ixs = arv[s], ixv[s]
            qx1 = jnp.maximum(xa, bx1)
            qy1 = jnp.maximum(ya, by1)
            qx2 = jnp.minimum(xb, bx2)
            qy2 = jnp.minimum(yb, by2)
            qin = jnp.maximum(qx2 - qx1, 0.0) * jnp.maximum(qy2 - qy1, 0.0)
            sup1 = qin / (ar + ar1 - qin + 1e-8) > IOU_THRESH
            ux1 = jnp.maximum(xa, cx1)
            uy1 = jnp.maximum(ya, cy1)
            ux2 = jnp.minimum(xb, cx2)
            uy2 = jnp.minimum(yb, cy2)
            uin = jnp.maximum(ux2 - ux1, 0.0) * jnp.maximum(uy2 - uy1, 0.0)
            sup2 = acc & (uin / (ar + ar2 - uin + 1e-8) > IOU_THRESH)
            w = jnp.where(sup1 | sup2, ninf, wkv[s])
            wkv[s] = w
            return upd2(w, ixs, st2)

        def sup_j(j, st2):
            sta, stb = st2
            sta = sup_one(pl.ds(j * 2 * L, L), sta)
            stb = sup_one(pl.ds(j * 2 * L + L, L), stb)
            return sta, stb

        sta, stb = lax.fori_loop(0, NSL // 2, sup_j, (st0, st0))
        nst = upd2(stb[0], stb[1], sta)
        nst = upd2(stb[2], stb[3], nst)

        z = jnp.float32(0.0)
        row1 = jnp.where(ii == 0, bx1, z)
        row1 = jnp.where(ii == 1, by1, row1)
        row1 = jnp.where(ii == 2, bx2, row1)
        row1 = jnp.where(ii == 3, by2, row1)
        row1 = jnp.where(ii == 4, bsc, row1)
        oiv = jnp.broadcast_to(oi, (L,))
        plsc.store_scatter(outv, [oiv * 5 + ii], row1, mask=ii < 5)
        row2 = jnp.where(ii == 0, cx1, z)
        row2 = jnp.where(ii == 1, cy1, row2)
        row2 = jnp.where(ii == 2, cx2, row2)
        row2 = jnp.where(ii == 3, cy2, row2)
        row2 = jnp.where(ii == 4, csc, row2)
        plsc.store_scatter(outv, [(oiv + 1) * 5 + ii], row2, mask=(ii < 5) & acc)

        sci[...] = jnp.where(acc, 1, 0)
        noi = oi + 1 + sci[...][0]
        return (noi, r + 1) + nst

    lax.while_loop(cond_fn, round_body,
                   (jnp.int32(0), jnp.int32(0)) + top0)

    @pl.when((sid == 0) & (cid == 0))
    def _():
        pltpu.sync_copy(outv, outh)


def _make_nms_sc():
    mesh = plsc.VectorSubcoreMesh(
        core_axis_name="c", subcore_axis_name="s", num_cores=2, num_subcores=NSUB
    )
    return pl.kernel(
        _nms_sc_body,
        out_type=jax.ShapeDtypeStruct((OUTPAD,), jnp.float32),
        mesh=mesh,
        compiler_params=pltpu.CompilerParams(needs_layout_passes=False),
        scratch_types=[
            pltpu.VMEM((SHARD,), jnp.float32),
            pltpu.VMEM((SHARD,), jnp.float32),
            pltpu.VMEM((SHARD,), jnp.float32),
            pltpu.VMEM((SHARD,), jnp.float32),
            pltpu.VMEM((SHARD,), jnp.float32),
            pltpu.VMEM((SHARD,), jnp.float32),
            pltpu.VMEM((SHARD,), jnp.float32),
            pltpu.VMEM((SHARD,), jnp.float32),
            pltpu.VMEM_SHARED((2 * NSUB * ROW,), jnp.float32),
            pltpu.VMEM((NSUB * ROW,), jnp.float32),
            pltpu.VMEM((ROW,), jnp.float32),
            pltpu.VMEM((L,), jnp.int32),
            pltpu.VMEM((OUTPAD,), jnp.float32),
        ],
    )


def kernel(boxes, scores):
    pad = NPAD - N
    x1 = jnp.pad(boxes[:, 0], (0, pad))
    y1 = jnp.pad(boxes[:, 1], (0, pad))
    x2 = jnp.pad(boxes[:, 2], (0, pad))
    y2 = jnp.pad(boxes[:, 3], (0, pad))
    sc = jnp.pad(scores, (0, pad))
    out = _make_nms_sc()(x1, y1, x2, y2, sc)
    return out[: MAX_OUT * 5].reshape(MAX_OUT, 5)
